# baseline (speedup 1.0000x reference)
import jax
import jax.numpy as jnp
from jax import lax
from jax.experimental import pallas as pl
from jax.experimental.pallas import tpu as pltpu
from jax.experimental.pallas import tpu_sc as plsc

VOCAB = 49408
NUM_POS = 77
EMBED_DIM = 768
BATCH = 4096

NUM_CORES = 2
NUM_SUBCORES = 16
NW = NUM_CORES * NUM_SUBCORES
ELEMS_PER_W = BATCH // NW
POS_PAD = 80
GA, GB = 48, 32
WB = 24
TAIL = NUM_POS - GA - WB
LANES = 16
DVEC = EMBED_DIM // LANES


def _sc_body(tokens_hbm, table_hbm, pos_hbm, out_hbm,
             idx0, idx1, pos_v, bufa, bufb, bufc,
             isem, gsema, gsemb, wsema, wsemb, wsemc):
    wid = lax.axis_index("s") * NUM_CORES + lax.axis_index("c")
    tok_base = wid * ELEMS_PER_W * POS_PAD

    pltpu.sync_copy(pos_hbm, pos_v)

    idxs = (idx0, idx1)

    def start_idx_load(e, p):
        pltpu.async_copy(tokens_hbm.at[pl.ds(tok_base + e * POS_PAD, POS_PAD)],
                         idxs[p], isem)

    def wait_idx_load(e, p):
        pltpu.make_async_copy(tokens_hbm.at[pl.ds(tok_base + e * POS_PAD, POS_PAD)],
                              idxs[p], isem).wait()

    def start_gather_a(p):
        pltpu.async_copy(table_hbm.at[idxs[p].at[pl.ds(0, GA)]], bufa, gsema)

    def wait_gather_a(p):
        pltpu.make_async_copy(table_hbm.at[idxs[p].at[pl.ds(0, GA)]], bufa, gsema).wait()

    def start_gather_b(p):
        pltpu.async_copy(table_hbm.at[idxs[p].at[pl.ds(GA, GB)]], bufb, gsemb)

    def wait_gather_b(p):
        pltpu.make_async_copy(table_hbm.at[idxs[p].at[pl.ds(GA, GB)]], bufb, gsemb).wait()

    def start_write_a(e):
        bb = wid * ELEMS_PER_W + e
        pltpu.async_copy(bufa, out_hbm.at[bb, pl.ds(0, GA)], wsema)

    def wait_write_a(e):
        bb = wid * ELEMS_PER_W + e
        pltpu.make_async_copy(bufa, out_hbm.at[bb, pl.ds(0, GA)], wsema).wait()

    def start_write_b(e):
        bb = wid * ELEMS_PER_W + e
        pltpu.async_copy(bufb.at[pl.ds(0, WB)], out_hbm.at[bb, pl.ds(GA, WB)], wsemb)

    def wait_write_b(e):
        bb = wid * ELEMS_PER_W + e
        pltpu.make_async_copy(bufb.at[pl.ds(0, WB)], out_hbm.at[bb, pl.ds(GA, WB)], wsemb).wait()

    def start_write_c(e):
        bb = wid * ELEMS_PER_W + e
        pltpu.async_copy(bufc, out_hbm.at[bb, pl.ds(GA + WB, TAIL)], wsemc)

    def wait_write_c(e):
        bb = wid * ELEMS_PER_W + e
        pltpu.make_async_copy(bufc, out_hbm.at[bb, pl.ds(GA + WB, TAIL)], wsemc).wait()

    def add_a():
        def row_body(r, _):
            pb = r * EMBED_DIM
            for j in range(DVEC):
                sl = pl.ds(j * LANES, LANES)
                bufa[r, sl] = bufa[r, sl] + pos_v[pl.ds(pb + j * LANES, LANES)]
            return 0
        lax.fori_loop(0, GA, row_body, 0)

    def add_b():
        def row_body(r, _):
            pb = (GA + r) * EMBED_DIM
            for j in range(DVEC):
                sl = pl.ds(j * LANES, LANES)
                bufb[r, sl] = bufb[r, sl] + pos_v[pl.ds(pb + j * LANES, LANES)]
            return 0
        lax.fori_loop(0, WB, row_body, 0)
        def tail_body(r, _):
            pb = (GA + WB + r) * EMBED_DIM
            for j in range(DVEC):
                sl = pl.ds(j * LANES, LANES)
                bufc[r, sl] = bufb[WB + r, sl] + pos_v[pl.ds(pb + j * LANES, LANES)]
            return 0
        lax.fori_loop(0, TAIL, tail_body, 0)

    start_idx_load(0, 0)
    wait_idx_load(0, 0)
    start_gather_a(0)

    def elem_body(t, _):
        for par in range(2):
            e = 2 * t + par

            @pl.when(e < ELEMS_PER_W - 1)
            def _():
                start_idx_load(e + 1, 1 - par)

            @pl.when(e > 0)
            def _():
                wait_write_b(e - 1)
                wait_write_c(e - 1)

            start_gather_b(par)
            wait_gather_a(par)
            add_a()
            start_write_a(e)
            wait_write_a(e)

            @pl.when(e < ELEMS_PER_W - 1)
            def _():
                wait_idx_load(e + 1, 1 - par)
                start_gather_a(1 - par)

            wait_gather_b(par)
            add_b()
            start_write_b(e)
            start_write_c(e)
        return 0

    lax.fori_loop(0, ELEMS_PER_W // 2, elem_body, 0)
    wait_write_b(ELEMS_PER_W - 1)
    wait_write_c(ELEMS_PER_W - 1)


def kernel(input_tokens, token_table, pos_table):
    tokens = input_tokens.astype(jnp.int32)
    tokens = jnp.pad(tokens, ((0, 0), (0, POS_PAD - NUM_POS)))
    tokens = tokens.reshape(-1)

    mesh = plsc.VectorSubcoreMesh(core_axis_name="c", subcore_axis_name="s")
    out = pl.kernel(
        _sc_body,
        out_type=jax.ShapeDtypeStruct((BATCH, NUM_POS, EMBED_DIM), jnp.float32),
        mesh=mesh,
        scratch_types=[
            pltpu.VMEM((POS_PAD,), jnp.int32),
            pltpu.VMEM((POS_PAD,), jnp.int32),
            pltpu.VMEM((NUM_POS * EMBED_DIM,), jnp.float32),
            pltpu.VMEM((GA, EMBED_DIM), jnp.float32),
            pltpu.VMEM((GB, EMBED_DIM), jnp.float32),
            pltpu.VMEM((TAIL, EMBED_DIM), jnp.float32),
            pltpu.SemaphoreType.DMA,
            pltpu.SemaphoreType.DMA,
            pltpu.SemaphoreType.DMA,
            pltpu.SemaphoreType.DMA,
            pltpu.SemaphoreType.DMA,
            pltpu.SemaphoreType.DMA,
        ],
    )(tokens, token_table, pos_table.reshape(-1))
    return out
